# Initial kernel scaffold; baseline (speedup 1.0000x reference)
import jax
import jax.numpy as jnp
from jax.experimental import pallas as pl


def kernel(x, loc_orig, idx_agg, agg_weight, conv_w, conv_b, skip_w, ln_g, ln_b, conf_w, conf_b):
    raise NotImplementedError("write your pallas kernel here")


# speedup vs baseline: 10.1478x; 10.1478x over previous
import functools
import math

import jax
import jax.numpy as jnp
from jax import lax
from jax.experimental import pallas as pl
from jax.experimental.pallas import tpu as pltpu
from jax.experimental.pallas import tpu_sc as plsc

B, N, C_IN, C_OUT = 2, 3136, 128, 256
H = W = 56
K = 5
NS = int(math.ceil(N * 0.25))
BM = 448
NB = N // BM
_INV_SQRT_C = 0.0625


def _dist_block(xr, xf, r2, c2):
    dot = lax.dot_general(xr, xf, (((1,), (1,)), ((), ())),
                          preferred_element_type=jnp.float32)
    d2 = r2 + c2 - 2.0 * dot
    return jnp.sqrt(jnp.maximum(d2, 0.0)) * _INV_SQRT_C


def _density_kernel(xr_ref, xf_ref, r2_ref, c2_ref, out_ref):
    dist = _dist_block(xr_ref[0], xf_ref[0], r2_ref[0], c2_ref[0])
    cols = lax.broadcasted_iota(jnp.int32, (BM, N), 1)
    mins = []
    for _ in range(K):
        m = jnp.min(dist, axis=1, keepdims=True)
        mins.append(m)
        first = jnp.min(jnp.where(dist == m, cols, N), axis=1, keepdims=True)
        dist = jnp.where(cols == first, jnp.inf, dist)
    out_ref[0] = jnp.concatenate(mins, axis=1)


def _parent_kernel(xr_ref, xf_ref, r2_ref, c2_ref, dr_ref, df_ref,
                   parent_ref, rowmax_ref):
    dist = _dist_block(xr_ref[0], xf_ref[0], r2_ref[0], c2_ref[0])
    mask = df_ref[0] > dr_ref[0]
    parent_ref[0] = jnp.min(jnp.where(mask, dist, jnp.inf), axis=1, keepdims=True)
    rowmax_ref[0] = jnp.max(dist, axis=1, keepdims=True)


def _assign_kernel(xt_ref, xc_ref, r2_ref, c2_ref, out_ref):
    dist = _dist_block(xt_ref[0], xc_ref[0], r2_ref[0], c2_ref[0])
    cols = lax.broadcasted_iota(jnp.int32, (BM, NS), 1)
    m = jnp.min(dist, axis=1, keepdims=True)
    out_ref[0] = jnp.min(jnp.where(dist == m, cols, NS), axis=1, keepdims=True)


def _density(xs, a2):
    out = pl.pallas_call(
        _density_kernel,
        grid=(B, NB),
        in_specs=[
            pl.BlockSpec((1, BM, C_OUT), lambda b, i: (b, i, 0)),
            pl.BlockSpec((1, N, C_OUT), lambda b, i: (b, 0, 0)),
            pl.BlockSpec((1, BM, 1), lambda b, i: (b, i, 0)),
            pl.BlockSpec((1, 1, N), lambda b, i: (b, 0, 0)),
        ],
        out_specs=pl.BlockSpec((1, BM, K), lambda b, i: (b, i, 0)),
        out_shape=jax.ShapeDtypeStruct((B, N, K), jnp.float32),
    )(xs, xs, a2.reshape(B, N, 1), a2.reshape(B, 1, N))
    return out


def _parent(xs, a2, density):
    parent, rowmax = pl.pallas_call(
        _parent_kernel,
        grid=(B, NB),
        in_specs=[
            pl.BlockSpec((1, BM, C_OUT), lambda b, i: (b, i, 0)),
            pl.BlockSpec((1, N, C_OUT), lambda b, i: (b, 0, 0)),
            pl.BlockSpec((1, BM, 1), lambda b, i: (b, i, 0)),
            pl.BlockSpec((1, 1, N), lambda b, i: (b, 0, 0)),
            pl.BlockSpec((1, BM, 1), lambda b, i: (b, i, 0)),
            pl.BlockSpec((1, 1, N), lambda b, i: (b, 0, 0)),
        ],
        out_specs=[
            pl.BlockSpec((1, BM, 1), lambda b, i: (b, i, 0)),
            pl.BlockSpec((1, BM, 1), lambda b, i: (b, i, 0)),
        ],
        out_shape=[
            jax.ShapeDtypeStruct((B, N, 1), jnp.float32),
            jax.ShapeDtypeStruct((B, N, 1), jnp.float32),
        ],
    )(xs, xs, a2.reshape(B, N, 1), a2.reshape(B, 1, N),
      density.reshape(B, N, 1), density.reshape(B, 1, N))
    return parent.reshape(B, N), rowmax.reshape(B, N)


def _assign(xs, centers, a2, c2g):
    out = pl.pallas_call(
        _assign_kernel,
        grid=(B, NB),
        in_specs=[
            pl.BlockSpec((1, BM, C_OUT), lambda b, i: (b, i, 0)),
            pl.BlockSpec((1, NS, C_OUT), lambda b, i: (b, 0, 0)),
            pl.BlockSpec((1, BM, 1), lambda b, i: (b, i, 0)),
            pl.BlockSpec((1, 1, NS), lambda b, i: (b, 0, 0)),
        ],
        out_specs=pl.BlockSpec((1, BM, 1), lambda b, i: (b, i, 0)),
        out_shape=jax.ShapeDtypeStruct((B, N, 1), jnp.int32),
    )(xs, centers, a2.reshape(B, N, 1), c2g.reshape(B, 1, NS))
    return out.reshape(B, N)


NT = 224
NCH = NT // 16
ACT = 14
CR = NS // ACT


def _sums_kernel(ic_ref, w_ref, xn_ref, y_ref, aw_ref):
    i = pl.program_id(1)
    cols = lax.broadcasted_iota(jnp.int32, (BM, NS), 1)
    ow = jnp.where(cols == ic_ref[0], w_ref[0], 0.0)
    y_part = lax.dot_general(ow, xn_ref[0], (((0,), (0,)), ((), ())),
                             precision=lax.Precision.HIGHEST,
                             preferred_element_type=jnp.float32)
    aw_part = jnp.sum(ow, axis=0, keepdims=True)

    @pl.when(i == 0)
    def _init():
        y_ref[0] = y_part
        aw_ref[0] = aw_part

    @pl.when(i > 0)
    def _acc():
        y_ref[0] = y_ref[0] + y_part
        aw_ref[0] = aw_ref[0] + aw_part


def _sums(idx_cluster, weight, xs):
    y, aw = pl.pallas_call(
        _sums_kernel,
        grid=(B, NB),
        in_specs=[
            pl.BlockSpec((1, BM, 1), lambda b, i: (b, i, 0)),
            pl.BlockSpec((1, BM, 1), lambda b, i: (b, i, 0)),
            pl.BlockSpec((1, BM, C_OUT), lambda b, i: (b, i, 0)),
        ],
        out_specs=[
            pl.BlockSpec((1, NS, C_OUT), lambda b, i: (b, 0, 0)),
            pl.BlockSpec((1, 1, NS), lambda b, i: (b, 0, 0)),
        ],
        out_shape=[
            jax.ShapeDtypeStruct((B, NS, C_OUT), jnp.float32),
            jax.ShapeDtypeStruct((B, 1, NS), jnp.float32),
        ],
    )(idx_cluster.reshape(B, N, 1), weight, xs)
    return y, aw.reshape(B, NS)


def _gather_kernel(aw_hbm, idxc_hbm, w_hbm, gidx_hbm, aggw_hbm,
                   iad_hbm, awd_hbm,
                   fidx_v, w_v, aw_local, nw_v,
                   nwtab, ictab, gidx_v, iad_v, awd_v, aggw_v,
                   nw_spmem):
    c = lax.axis_index("c")
    sid = lax.axis_index("s")
    base = c * N + sid * NT

    @pl.when(sid < ACT)
    def _phase_a():
        pltpu.sync_copy(idxc_hbm.at[pl.ds(base, NT)], fidx_v)
        pltpu.sync_copy(w_hbm.at[pl.ds(base, NT)], w_v)
        pltpu.sync_copy(aw_hbm.at[pl.ds(c * NS, NS)], aw_local)
        for i in range(NCH):
            idx16 = fidx_v[pl.ds(16 * i, 16)]
            awg = plsc.load_gather(aw_local, [idx16])
            nw_v[pl.ds(16 * i, 16)] = w_v[pl.ds(16 * i, 16)] / (awg + 1e-6)
        pltpu.sync_copy(nw_v, nw_spmem.at[pl.ds(sid * NT, NT)])
    plsc.subcore_barrier()

    @pl.when(sid < ACT)
    def _phase_b():
        pltpu.sync_copy(nw_spmem, nwtab)
        pltpu.sync_copy(idxc_hbm.at[pl.ds(c * N, N)], ictab)
        pltpu.sync_copy(gidx_hbm.at[pl.ds(base, NT)], gidx_v)
        pltpu.sync_copy(aggw_hbm.at[pl.ds(base, NT)], aggw_v)
        for i in range(NCH):
            g16 = gidx_v[pl.ds(16 * i, 16)]
            iad_v[pl.ds(16 * i, 16)] = plsc.load_gather(ictab, [g16])
            wt16 = plsc.load_gather(nwtab, [g16])
            awd_v[pl.ds(16 * i, 16)] = aggw_v[pl.ds(16 * i, 16)] * wt16
        pltpu.sync_copy(iad_v, iad_hbm.at[pl.ds(base, NT)])
        pltpu.sync_copy(awd_v, awd_hbm.at[pl.ds(base, NT)])


def _gathers(all_w, idx_cluster, weight, idx_agg, agg_weight):
    mesh = plsc.VectorSubcoreMesh(core_axis_name="c", subcore_axis_name="s")
    run = pl.kernel(
        _gather_kernel,
        mesh=mesh,
        compiler_params=pltpu.CompilerParams(needs_layout_passes=False),
        out_type=[
            jax.ShapeDtypeStruct((B * N,), jnp.int32),
            jax.ShapeDtypeStruct((B * N,), jnp.float32),
        ],
        scratch_types=[
            pltpu.VMEM((NT,), jnp.int32),
            pltpu.VMEM((NT,), jnp.float32),
            pltpu.VMEM((NS,), jnp.float32),
            pltpu.VMEM((NT,), jnp.float32),
            pltpu.VMEM((N,), jnp.float32),
            pltpu.VMEM((N,), jnp.int32),
            pltpu.VMEM((NT,), jnp.int32),
            pltpu.VMEM((NT,), jnp.int32),
            pltpu.VMEM((NT,), jnp.float32),
            pltpu.VMEM((NT,), jnp.float32),
            pltpu.VMEM_SHARED((N,), jnp.float32),
        ],
    )
    return run(all_w.reshape(B * NS), idx_cluster.reshape(B * N),
               weight.reshape(B * N), idx_agg.reshape(B * N),
               agg_weight.reshape(B * N))


def _copy_kernel(src_ref, dst_ref):
    dst_ref[...] = src_ref[...]


def _materialize(x):
    return pl.pallas_call(
        _copy_kernel,
        grid=(B,),
        in_specs=[pl.BlockSpec((1, N, C_OUT), lambda b: (b, 0, 0))],
        out_specs=pl.BlockSpec((1, N, C_OUT), lambda b: (b, 0, 0)),
        out_shape=jax.ShapeDtypeStruct((B, N, C_OUT), jnp.float32),
    )(x)


def _layernorm(x, g, b, eps=1e-5):
    m = jnp.mean(x, axis=-1, keepdims=True)
    v = jnp.mean((x - m) ** 2, axis=-1, keepdims=True)
    return (x - m) / jnp.sqrt(v + eps) * g + b


def kernel(x, loc_orig, idx_agg, agg_weight, conv_w, conv_b, skip_w, ln_g, ln_b, conf_w, conf_b):
    x_map = x.reshape(B, H, W, C_IN).transpose(0, 3, 1, 2)
    x_map = lax.conv_general_dilated(x_map, conv_w, (2, 2), ((1, 1), (1, 1)),
                                     dimension_numbers=('NCHW', 'OIHW', 'NCHW'))
    x_map = x_map + conv_b[None, :, None, None]
    Hd, Wd = x_map.shape[2], x_map.shape[3]
    loc = 0.5 * (loc_orig + 1.0) * jnp.array([Wd, Hd], dtype=jnp.float32) - 0.5
    xi = jnp.clip(jnp.round(loc[..., 0]).astype(jnp.int32), 0, Wd - 1)
    yi = jnp.clip(jnp.round(loc[..., 1]).astype(jnp.int32), 0, Hd - 1)
    idx_hw = yi * Wd + xi
    feat = x_map.reshape(B, C_OUT, Hd * Wd).transpose(0, 2, 1)
    gathered = jnp.take_along_axis(feat, idx_hw[..., None], axis=1)

    def agg_one(g, idx, w):
        num = jax.ops.segment_sum(g * w, idx, num_segments=N)
        den = jax.ops.segment_sum(w, idx, num_segments=N)
        return num / (den + 1e-6)

    tok = jax.vmap(agg_one)(gathered, idx_agg, agg_weight)
    xn = tok + jnp.einsum('bnc,dc->bnd', x, skip_w)
    xn = _layernorm(xn, ln_g, ln_b)
    xn = _materialize(xn)
    conf = jnp.einsum('bnd,od->bno', xn, conf_w) + conf_b
    weight = jnp.exp(conf)

    a2 = jnp.sum(xn * xn, axis=-1)
    dist_nearest = _density(xn, a2)
    density = jnp.exp(-jnp.mean(dist_nearest ** 2, axis=-1))
    density = density + jax.random.uniform(jax.random.key(1), density.shape,
                                           dtype=jnp.float32) * 1e-6
    parent_raw, rowmax = _parent(xn, a2, density)
    dist_max = jnp.max(rowmax, axis=-1, keepdims=True)
    dist_parent = jnp.where(jnp.isinf(parent_raw), dist_max, parent_raw)
    score = dist_parent * density
    _, index_down = lax.top_k(score, NS)
    centers = jnp.take_along_axis(xn, index_down[..., None], axis=1)
    c2g = jnp.take_along_axis(a2, index_down, axis=1)
    idx_cluster = _assign(xn, centers, a2, c2g)
    idx_cluster = idx_cluster.at[jnp.arange(B)[:, None], index_down].set(
        jnp.broadcast_to(jnp.arange(NS, dtype=jnp.int32)[None, :], (B, NS)))

    y, all_w = _sums(idx_cluster, weight, xn)
    iad, awd = _gathers(all_w, idx_cluster, weight, idx_agg, agg_weight)
    x_down = y / (all_w + 1e-6)[..., None]
    idx_agg_down = iad.reshape(B, N)
    agg_weight_down = awd.reshape(B, N, 1)
    agg_weight_down = agg_weight_down / jnp.max(agg_weight_down, axis=1, keepdims=True)
    return x_down, idx_agg_down, agg_weight_down
